# Initial kernel scaffold; baseline (speedup 1.0000x reference)
import jax
import jax.numpy as jnp
from jax.experimental import pallas as pl


def kernel(h, gate_w, wg, wu, wd):
    raise NotImplementedError("write your pallas kernel here")


# speedup vs baseline: 3.5388x; 3.5388x over previous
import functools

import jax
import jax.numpy as jnp
from jax import lax
from jax.experimental import pallas as pl
from jax.experimental.pallas import tpu as pltpu
from jax.experimental.pallas import tpu_sc as plsc

BB, LL, DD = 2, 2048, 768
INTER = 2048
NE = 64
TOPK = 2
LB_ALPHA = 0.01
TT = BB * LL
AA = TT * TOPK
G = 128
NB = AA // G + NE
PP = NB * G
BT = 512

_SC_INFO = plsc.get_sparse_core_info()
_NC, _NS = _SC_INFO.num_cores, _SC_INFO.num_subcores
_NW = _NC * _NS


def _gate_body(x_ref, gw_ref, a1_ref, a2_ref, w1_ref, w2_ref, route_ref,
               psum_ref):
    i = pl.program_id(0)
    x = x_ref[...]
    gw = gw_ref[...]
    logits = lax.dot_general(x, gw, (((1,), (1,)), ((), ())),
                             preferred_element_type=jnp.float32)
    iota = lax.broadcasted_iota(jnp.int32, (BT, NE), 1)
    m1 = jnp.max(logits, axis=1, keepdims=True)
    a1 = jnp.min(jnp.where(logits == m1, iota, NE), axis=1, keepdims=True)
    masked = jnp.where(iota == a1, -jnp.inf, logits)
    m2 = jnp.max(masked, axis=1, keepdims=True)
    a2 = jnp.min(jnp.where(masked == m2, iota, NE), axis=1, keepdims=True)
    w1 = jax.nn.sigmoid(m1 - m2)
    w2 = 1.0 - w1
    route = jnp.where(iota == a1, w1, 0.0) + jnp.where(iota == a2, w2, 0.0)
    route_ref[...] = route
    a1_ref[...] = a1.reshape(BT)
    a2_ref[...] = a2.reshape(BT)
    w1_ref[...] = w1.reshape(BT)
    w2_ref[...] = w2.reshape(BT)

    @pl.when(i == 0)
    def _():
        psum_ref[...] = jnp.zeros_like(psum_ref)

    psum_ref[...] += jnp.sum(route, axis=0)


def _gating(hf, gate_w):
    return pl.pallas_call(
        _gate_body,
        grid=(TT // BT,),
        in_specs=[
            pl.BlockSpec((BT, DD), lambda i: (i, 0)),
            pl.BlockSpec((NE, DD), lambda i: (0, 0)),
        ],
        out_specs=[
            pl.BlockSpec((BT,), lambda i: (i,)),
            pl.BlockSpec((BT,), lambda i: (i,)),
            pl.BlockSpec((BT,), lambda i: (i,)),
            pl.BlockSpec((BT,), lambda i: (i,)),
            pl.BlockSpec((BT, NE), lambda i: (i, 0)),
            pl.BlockSpec((NE,), lambda i: (0,)),
        ],
        out_shape=[
            jax.ShapeDtypeStruct((TT,), jnp.int32),
            jax.ShapeDtypeStruct((TT,), jnp.int32),
            jax.ShapeDtypeStruct((TT,), jnp.float32),
            jax.ShapeDtypeStruct((TT,), jnp.float32),
            jax.ShapeDtypeStruct((TT, NE), jnp.float32),
            jax.ShapeDtypeStruct((NE,), jnp.float32),
        ],
        compiler_params=pltpu.CompilerParams(
            dimension_semantics=("arbitrary",)),
    )(hf, gate_w)


def _make_sc_gather(n_rows, n_cols, chunk):
    n_ch = n_rows // (_NW * chunk)
    per_w = n_rows // _NW
    mesh = plsc.VectorSubcoreMesh(core_axis_name="c", subcore_axis_name="s")

    @functools.partial(
        pl.kernel, mesh=mesh,
        out_type=jax.ShapeDtypeStruct((n_rows, n_cols), jnp.float32),
        scratch_types=[
            pltpu.VMEM((chunk,), jnp.int32),
            pltpu.VMEM((chunk, n_cols), jnp.float32),
            pltpu.SemaphoreType.DMA,
        ],
    )
    def gather_k(table_hbm, idx_hbm, out_hbm, idx_v, rows_v, sem):
        wid = lax.axis_index("s") * _NC + lax.axis_index("c")
        base = wid * per_w

        def body(c, carry):
            off = base + c * chunk
            pltpu.sync_copy(idx_hbm.at[pl.ds(off, chunk)], idx_v)
            pltpu.async_copy(table_hbm.at[idx_v], rows_v, sem).wait()
            pltpu.sync_copy(rows_v, out_hbm.at[pl.ds(off, chunk)])
            return carry

        lax.fori_loop(0, n_ch, body, 0)

    return gather_k


def _make_sc_combine(n_cols, chunk):
    per_w = TT // _NW
    n_ch = per_w // chunk
    mesh = plsc.VectorSubcoreMesh(core_axis_name="c", subcore_axis_name="s")

    @functools.partial(
        pl.kernel, mesh=mesh,
        out_type=jax.ShapeDtypeStruct((TT, n_cols), jnp.float32),
        scratch_types=[
            pltpu.VMEM((chunk,), jnp.int32),
            pltpu.VMEM((chunk,), jnp.int32),
            pltpu.VMEM((chunk, n_cols), jnp.float32),
            pltpu.VMEM((chunk, n_cols), jnp.float32),
            pltpu.SemaphoreType.DMA,
        ],
    )
    def combine_k(y_hbm, p0_hbm, p1_hbm, out_hbm, i0_v, i1_v, buf_a, buf_b,
                  sem):
        wid = lax.axis_index("s") * _NC + lax.axis_index("c")
        base = wid * per_w

        def body(c, carry):
            off = base + c * chunk
            pltpu.sync_copy(p0_hbm.at[pl.ds(off, chunk)], i0_v)
            pltpu.sync_copy(p1_hbm.at[pl.ds(off, chunk)], i1_v)
            pltpu.async_copy(y_hbm.at[i0_v], buf_a, sem).wait()
            pltpu.async_copy(y_hbm.at[i1_v], buf_b, sem).wait()

            def row(r, rc):
                for j in range(n_cols // 16):
                    sl = pl.ds(j * 16, 16)
                    buf_a[r, sl] = buf_a[r, sl] + buf_b[r, sl]
                return rc

            lax.fori_loop(0, chunk, row, 0)
            pltpu.sync_copy(buf_a, out_hbm.at[pl.ds(off, chunk)])
            return carry

        lax.fori_loop(0, n_ch, body, 0)

    return combine_k


def _ffn_body(be_ref, x_ref, wsc_ref, wg_ref, wu_ref, wd_ref, o_ref):
    x = x_ref[...]
    g = lax.dot_general(x, wg_ref[0], (((1,), (1,)), ((), ())),
                        preferred_element_type=jnp.float32)
    u = lax.dot_general(x, wu_ref[0], (((1,), (1,)), ((), ())),
                        preferred_element_type=jnp.float32)
    hmid = g * jax.nn.sigmoid(g) * u
    y = lax.dot_general(hmid, wd_ref[0], (((1,), (1,)), ((), ())),
                        preferred_element_type=jnp.float32)
    o_ref[...] = y * wsc_ref[0, 0][:, None]


def _grouped_ffn(blk_expert, xg, w_pad3, wg, wu, wd):
    grid_spec = pltpu.PrefetchScalarGridSpec(
        num_scalar_prefetch=1,
        grid=(NB,),
        in_specs=[
            pl.BlockSpec((G, DD), lambda b, be: (b, 0)),
            pl.BlockSpec((1, 1, G), lambda b, be: (b, 0, 0)),
            pl.BlockSpec((1, INTER, DD), lambda b, be: (be[b], 0, 0)),
            pl.BlockSpec((1, INTER, DD), lambda b, be: (be[b], 0, 0)),
            pl.BlockSpec((1, DD, INTER), lambda b, be: (be[b], 0, 0)),
        ],
        out_specs=pl.BlockSpec((G, DD), lambda b, be: (b, 0)),
    )
    return pl.pallas_call(
        _ffn_body,
        grid_spec=grid_spec,
        out_shape=jax.ShapeDtypeStruct((PP, DD), jnp.float32),
        compiler_params=pltpu.CompilerParams(
            dimension_semantics=("arbitrary",)),
    )(blk_expert, xg, w_pad3, wg, wu, wd)


def kernel(h, gate_w, wg, wu, wd):
    hf = h.reshape(TT, DD)
    a1, a2, w1, w2, route, psum = _gating(hf, gate_w)

    e_flat = jnp.stack([a1, a2], axis=1).reshape(AA)
    w_flat = jnp.stack([w1, w2], axis=1).reshape(AA)
    t_flat = (jnp.arange(AA, dtype=jnp.int32) // TOPK).astype(jnp.int32)
    order = jnp.argsort(e_flat)
    se = e_flat[order]
    counts = jnp.bincount(e_flat, length=NE)
    bpe = (counts + G - 1) // G
    cumb = jnp.cumsum(bpe)
    pstart = (cumb - bpe) * G
    starts = jnp.cumsum(counts) - counts
    rank = jnp.arange(AA, dtype=jnp.int32) - starts[se]
    dest = (pstart[se] + rank).astype(jnp.int32)
    tok_pad = jnp.zeros((PP,), jnp.int32).at[dest].set(t_flat[order])
    w_pad = jnp.zeros((PP,), jnp.float32).at[dest].set(w_flat[order])
    blk_expert = jnp.clip(
        jnp.searchsorted(cumb, jnp.arange(NB), side="right"), 0,
        NE - 1).astype(jnp.int32)
    pos = jnp.zeros((AA,), jnp.int32).at[order].set(dest)
    pos0 = pos[0::2]
    pos1 = pos[1::2]

    xg = _make_sc_gather(PP, DD, 128)(hf, tok_pad)

    w_pad3 = w_pad.reshape(NB, 1, G)
    y_pad = _grouped_ffn(blk_expert, xg, w_pad3, wg, wu, wd)

    out_flat = _make_sc_combine(DD, 64)(y_pad, pos0, pos1)

    results = out_flat.reshape(BB, LL, DD)
    p_i = psum / TT
    lb_loss = LB_ALPHA * jnp.sum(p_i * p_i)
    route_probs_all = route.reshape(BB, LL, NE)
    return results, lb_loss, route_probs_all
